# Initial kernel scaffold; baseline (speedup 1.0000x reference)
import jax
import jax.numpy as jnp
from jax.experimental import pallas as pl


def kernel(op, a, b, op_embed, W_in, b_in, ln_in_g, ln_in_b, router_W, router_b, W1, b1, W2, b2, ln_g, ln_b, Wh1, bh1, Wh2, bh2):
    raise NotImplementedError("write your pallas kernel here")


# speedup vs baseline: 2.6619x; 2.6619x over previous
import functools

import numpy as np

import jax
import jax.numpy as jnp
from jax import lax
from jax.experimental import pallas as pl
from jax.experimental.pallas import tpu as pltpu
from jax.experimental.pallas import tpu_sc as plsc

B = 2048
D = 1024
T = 8
DFF = 1024
NF = 6
VR = 16
NOPS = 2

TB = 256
NTB = B // TB
BLK = 256
NPAD = B + T * BLK
NB = NPAD // BLK
NW = 32
CHUNK = B // NW
LANES = 16


def _gelu(x):
    return 0.5 * x * (1.0 + lax.erf(x * np.float32(1.0 / np.sqrt(2.0))))


def _layernorm(x, g, b):
    m = jnp.mean(x, axis=-1, keepdims=True)
    d = x - m
    v = jnp.mean(d * d, axis=-1, keepdims=True)
    return d / jnp.sqrt(v + 1e-5) * g + b


def _frontend_body(op_ref, a_ref, b_ref, emb_ref, wop_ref, wa_ref, wb_ref,
                   bin_ref, lng_ref, lnb_ref, rw_ref, rb_ref,
                   x_ref, tidx_ref, gate_ref, rank_ref, cnt_ref, psum_ref,
                   padoff_ref, eid_ref, valid_ref, run_cnt, acc_psum):
    i = pl.program_id(0)

    @pl.when(i == 0)
    def _():
        run_cnt[...] = jnp.zeros((1, 16), jnp.float32)
        acc_psum[...] = jnp.zeros((1, 16), jnp.float32)

    opv = op_ref[0, 0, :]
    av = a_ref[0, 0, :]
    bv = b_ref[0, 0, :]

    bf16 = jnp.bfloat16
    f32 = jnp.float32

    ci = lax.broadcasted_iota(jnp.int32, (TB, 16), 1)
    k = jnp.where(ci < 6, ci, ci - 6)
    freq = jnp.left_shift(1, k).astype(f32)
    ang_a = av.astype(f32)[:, None] * (2.0 * np.pi / VR) * freq
    ang_b = bv.astype(f32)[:, None] * (2.0 * np.pi / VR) * freq
    fa = jnp.where(ci < 6, jnp.sin(ang_a),
                   jnp.where(ci < 12, jnp.cos(ang_a), 0.0))
    fb = jnp.where(ci < 6, jnp.sin(ang_b),
                   jnp.where(ci < 12, jnp.cos(ang_b), 0.0))

    oh_op = (lax.broadcasted_iota(jnp.int32, (TB, 16), 1)
             == opv[:, None]).astype(bf16)
    feat_op = jnp.dot(oh_op, emb_ref[...].astype(bf16),
                      preferred_element_type=f32)

    x0 = (jnp.dot(feat_op.astype(bf16), wop_ref[...].astype(bf16),
                  preferred_element_type=f32)
          + jnp.dot(fa.astype(bf16), wa_ref[...].astype(bf16),
                    preferred_element_type=f32)
          + jnp.dot(fb.astype(bf16), wb_ref[...].astype(bf16),
                    preferred_element_type=f32)
          + bin_ref[...])
    x = _layernorm(x0, lng_ref[...], lnb_ref[...])
    x_ref[...] = x

    rl = jnp.dot(x.astype(bf16), rw_ref[...].astype(bf16),
                 preferred_element_type=f32) + rb_ref[...]
    lane = lax.broadcasted_iota(jnp.int32, (TB, 16), 1)
    rl = jnp.where(lane < T, rl, -1e30)
    mx = jnp.max(rl, axis=1, keepdims=True)
    ex = jnp.exp(rl - mx)
    probs = ex / jnp.sum(ex, axis=1, keepdims=True)
    gate = jnp.max(probs, axis=1)
    ismax = probs == gate[:, None]
    tidx = jnp.min(jnp.where(ismax, lane, T), axis=1)
    gate_ref[0, 0, :] = gate
    tidx_ref[0, 0, :] = tidx

    oh_e = (lane == tidx[:, None]).astype(jnp.float32)
    tri = (lax.broadcasted_iota(jnp.int32, (TB, TB), 0)
           > lax.broadcasted_iota(jnp.int32, (TB, TB), 1)).astype(jnp.float32)
    cum_excl = jnp.dot(tri, oh_e, preferred_element_type=jnp.float32)
    rank = jnp.sum((cum_excl + run_cnt[...]) * oh_e, axis=1)
    rank_ref[0, 0, :] = rank.astype(jnp.int32)

    run_cnt[...] = run_cnt[...] + jnp.sum(oh_e, axis=0, keepdims=True)
    acc_psum[...] = acc_psum[...] + jnp.sum(probs, axis=0, keepdims=True)

    @pl.when(i == NTB - 1)
    def _():
        cnt_ref[...] = run_cnt[...]
        psum_ref[...] = acc_psum[...]
        cntf = run_cnt[...]
        nblk = jnp.floor((cntf + (BLK - 1)) * (1.0 / BLK))
        ltri = (lax.broadcasted_iota(jnp.int32, (16, 16), 0)
                <= lax.broadcasted_iota(jnp.int32, (16, 16), 1)
                ).astype(jnp.float32)
        incl = jnp.dot(nblk, ltri, preferred_element_type=jnp.float32)
        padoff_ref[...] = ((incl - nblk) * BLK).astype(jnp.int32)
        jf = lax.broadcasted_iota(jnp.int32, (16, 16), 0).astype(jnp.float32)
        cmp = (jf >= incl).astype(jnp.float32)
        eid_ref[...] = jnp.minimum(
            jnp.sum(cmp, axis=1, keepdims=True), T - 1).astype(jnp.int32)
        valid_ref[...] = (1.0 - cmp[:, T - 1:T]).astype(jnp.int32)


def _frontend(op3, a3, b3, emb16, wop, wa, wb, bin2, lnig2, lnib2, rw16, rb16):
    f32 = jnp.float32
    return pl.pallas_call(
        _frontend_body,
        grid=(NTB,),
        in_specs=[
            pl.BlockSpec((1, 1, TB), lambda i: (i, 0, 0)),
            pl.BlockSpec((1, 1, TB), lambda i: (i, 0, 0)),
            pl.BlockSpec((1, 1, TB), lambda i: (i, 0, 0)),
            pl.BlockSpec((16, D // 4), lambda i: (0, 0)),
            pl.BlockSpec((D // 4, D), lambda i: (0, 0)),
            pl.BlockSpec((16, D), lambda i: (0, 0)),
            pl.BlockSpec((16, D), lambda i: (0, 0)),
            pl.BlockSpec((1, D), lambda i: (0, 0)),
            pl.BlockSpec((1, D), lambda i: (0, 0)),
            pl.BlockSpec((1, D), lambda i: (0, 0)),
            pl.BlockSpec((D, 16), lambda i: (0, 0)),
            pl.BlockSpec((1, 16), lambda i: (0, 0)),
        ],
        out_specs=[
            pl.BlockSpec((TB, D), lambda i: (i, 0)),
            pl.BlockSpec((1, 1, TB), lambda i: (i, 0, 0)),
            pl.BlockSpec((1, 1, TB), lambda i: (i, 0, 0)),
            pl.BlockSpec((1, 1, TB), lambda i: (i, 0, 0)),
            pl.BlockSpec((1, 16), lambda i: (0, 0)),
            pl.BlockSpec((1, 16), lambda i: (0, 0)),
            pl.BlockSpec((1, 16), lambda i: (0, 0)),
            pl.BlockSpec((16, 1), lambda i: (0, 0)),
            pl.BlockSpec((16, 1), lambda i: (0, 0)),
        ],
        out_shape=[
            jax.ShapeDtypeStruct((B, D), f32),
            jax.ShapeDtypeStruct((NTB, 1, TB), jnp.int32),
            jax.ShapeDtypeStruct((NTB, 1, TB), f32),
            jax.ShapeDtypeStruct((NTB, 1, TB), jnp.int32),
            jax.ShapeDtypeStruct((1, 16), f32),
            jax.ShapeDtypeStruct((1, 16), f32),
            jax.ShapeDtypeStruct((1, 16), jnp.int32),
            jax.ShapeDtypeStruct((16, 1), jnp.int32),
            jax.ShapeDtypeStruct((16, 1), jnp.int32),
        ],
        scratch_shapes=[
            pltpu.VMEM((1, 16), f32),
            pltpu.VMEM((1, 16), f32),
        ],
    )(op3, a3, b3, emb16, wop, wa, wb, bin2, lnig2, lnib2, rw16, rb16)


def _dst_body(tidx_ref, rank_ref, padoff_ref, dst_ref):
    acc = rank_ref[...]
    e_mat = tidx_ref[...]
    for e in range(T):
        acc = acc + jnp.where(e_mat == e, padoff_ref[e], 0)
    dst_ref[...] = acc


def _dst_compute(tidx2, rank2, padoff16):
    return pl.pallas_call(
        _dst_body,
        in_specs=[
            pl.BlockSpec(memory_space=pltpu.MemorySpace.VMEM),
            pl.BlockSpec(memory_space=pltpu.MemorySpace.VMEM),
            pl.BlockSpec(memory_space=pltpu.MemorySpace.SMEM),
        ],
        out_specs=pl.BlockSpec(memory_space=pltpu.MemorySpace.VMEM),
        out_shape=jax.ShapeDtypeStruct((16, 128), jnp.int32),
    )(tidx2, rank2, padoff16)


def _dispatch_body(x_hbm, dst_hbm, xs_hbm, dst_v, xv, sem):
    wid = lax.axis_index("s") * 2 + lax.axis_index("c")
    base = wid * CHUNK
    pltpu.sync_copy(dst_hbm.at[pl.ds(base, CHUNK)], dst_v)
    pltpu.sync_copy(x_hbm.at[pl.ds(base, CHUNK), :], xv)
    pltpu.async_copy(xv, xs_hbm.at[dst_v], sem).wait()


def _dispatch(x, dst):
    return pl.kernel(
        _dispatch_body,
        out_type=jax.ShapeDtypeStruct((NPAD, D), jnp.float32),
        mesh=plsc.VectorSubcoreMesh(core_axis_name="c", subcore_axis_name="s", num_cores=2, num_subcores=16),
        scratch_types=[
            pltpu.VMEM((CHUNK,), jnp.int32),
            pltpu.VMEM((CHUNK, D), jnp.float32),
            pltpu.SemaphoreType.DMA,
        ],
    )(x, dst)


def _ffn_body(eid_ref, valid_ref, xs_ref, w1_ref, b1_ref, w2_ref, b2_ref,
              ys_ref):
    j = pl.program_id(0)

    @pl.when(valid_ref[j] == 1)
    def _():
        x = xs_ref[...]
        h = jnp.dot(x, w1_ref[0], preferred_element_type=jnp.float32) + b1_ref[0]
        h = _gelu(h)
        y = jnp.dot(h, w2_ref[0], preferred_element_type=jnp.float32) + b2_ref[0]
        ys_ref[...] = y


def _ffn(eid, valid, xs, W1, b1, W2, b2):
    grid_spec = pltpu.PrefetchScalarGridSpec(
        num_scalar_prefetch=2,
        grid=(NB,),
        in_specs=[
            pl.BlockSpec((BLK, D), lambda j, eid_ref, valid_ref: (j, 0)),
            pl.BlockSpec((1, D, DFF),
                         lambda j, eid_ref, valid_ref: (eid_ref[j], 0, 0)),
            pl.BlockSpec((1, 1, DFF),
                         lambda j, eid_ref, valid_ref: (eid_ref[j], 0, 0)),
            pl.BlockSpec((1, DFF, D),
                         lambda j, eid_ref, valid_ref: (eid_ref[j], 0, 0)),
            pl.BlockSpec((1, 1, D),
                         lambda j, eid_ref, valid_ref: (eid_ref[j], 0, 0)),
        ],
        out_specs=pl.BlockSpec((BLK, D), lambda j, eid_ref, valid_ref: (j, 0)),
    )
    return pl.pallas_call(
        _ffn_body,
        grid_spec=grid_spec,
        out_shape=jax.ShapeDtypeStruct((NPAD, D), jnp.float32),
    )(eid, valid, xs, W1, b1.reshape(T, 1, DFF), W2, b2.reshape(T, 1, D))


def _combine_body(ys_hbm, dst_hbm, y_hbm, dst_v, yv, sem):
    wid = lax.axis_index("s") * 2 + lax.axis_index("c")
    base = wid * CHUNK
    pltpu.sync_copy(dst_hbm.at[pl.ds(base, CHUNK)], dst_v)
    pltpu.async_copy(ys_hbm.at[dst_v], yv, sem).wait()
    pltpu.sync_copy(yv, y_hbm.at[pl.ds(base, CHUNK), :])


def _combine(ys, dst):
    return pl.kernel(
        _combine_body,
        out_type=jax.ShapeDtypeStruct((B, D), jnp.float32),
        mesh=plsc.VectorSubcoreMesh(core_axis_name="c", subcore_axis_name="s", num_cores=2, num_subcores=16),
        scratch_types=[
            pltpu.VMEM((CHUNK,), jnp.int32),
            pltpu.VMEM((CHUNK, D), jnp.float32),
            pltpu.SemaphoreType.DMA,
        ],
    )(ys, dst)


def _tail_body(x_ref, y_ref, gate_ref, lng_ref, lnb_ref, wh1_ref, bh1_ref,
               wh2_ref, bh2_ref, cnt_ref, psum_ref, logits_ref, aux_ref):
    i = pl.program_id(0)
    g = gate_ref[0, 0, :][:, None]
    z = _layernorm(x_ref[...] + y_ref[...] * g, lng_ref[...], lnb_ref[...])
    h = _gelu(jnp.dot(z, wh1_ref[...], preferred_element_type=jnp.float32)
              + bh1_ref[...])
    logits_ref[0] = jnp.dot(h, wh2_ref[...],
                            preferred_element_type=jnp.float32) + bh2_ref[...]

    @pl.when(i == 0)
    def _():
        aux_ref[...] = (T / (B * B)) * jnp.sum(
            cnt_ref[...] * psum_ref[...], axis=1, keepdims=True)


def _tail(x, y, gate3, lng2, lnb2, Wh1, bh1_2, Wh2, bh2_2, cnt, psum):
    f32 = jnp.float32
    return pl.pallas_call(
        _tail_body,
        grid=(NTB,),
        in_specs=[
            pl.BlockSpec((TB, D), lambda i: (i, 0)),
            pl.BlockSpec((TB, D), lambda i: (i, 0)),
            pl.BlockSpec((1, 1, TB), lambda i: (i, 0, 0)),
            pl.BlockSpec((1, D), lambda i: (0, 0)),
            pl.BlockSpec((1, D), lambda i: (0, 0)),
            pl.BlockSpec((D, D // 2), lambda i: (0, 0)),
            pl.BlockSpec((1, D // 2), lambda i: (0, 0)),
            pl.BlockSpec((D // 2, 6), lambda i: (0, 0)),
            pl.BlockSpec((1, 6), lambda i: (0, 0)),
            pl.BlockSpec((1, 16), lambda i: (0, 0)),
            pl.BlockSpec((1, 16), lambda i: (0, 0)),
        ],
        out_specs=[
            pl.BlockSpec((1, TB, 6), lambda i: (i, 0, 0)),
            pl.BlockSpec((1, 1), lambda i: (0, 0)),
        ],
        out_shape=[
            jax.ShapeDtypeStruct((NTB, TB, 6), f32),
            jax.ShapeDtypeStruct((1, 1), f32),
        ],
    )(x, y, gate3, lng2, lnb2, Wh1, bh1_2, Wh2, bh2_2, cnt, psum)


def kernel(op, a, b, op_embed, W_in, b_in, ln_in_g, ln_in_b, router_W,
           router_b, W1, b1, W2, b2, ln_g, ln_b, Wh1, bh1, Wh2, bh2):
    i32 = jnp.int32
    op3 = op.astype(i32).reshape(NTB, 1, TB)
    a3 = a.astype(i32).reshape(NTB, 1, TB)
    b3 = b.astype(i32).reshape(NTB, 1, TB)
    emb16 = jnp.pad(op_embed, ((0, 16 - NOPS), (0, 0)))
    wop = W_in[:D // 4]
    wa = jnp.pad(W_in[D // 4:D // 4 + 2 * NF], ((0, 16 - 2 * NF), (0, 0)))
    wb = jnp.pad(W_in[D // 4 + 2 * NF:], ((0, 16 - 2 * NF), (0, 0)))
    rw16 = jnp.pad(router_W, ((0, 0), (0, 16 - T)))
    rb16 = jnp.pad(router_b, (0, 16 - T)).reshape(1, 16)

    (x, tidx3, gate3, rank3, cnt, psum, padoff, eid2, valid2) = _frontend(
        op3, a3, b3, emb16, wop, wa, wb, b_in.reshape(1, D),
        ln_in_g.reshape(1, D), ln_in_b.reshape(1, D), rw16, rb16)

    tidx = tidx3.reshape(B)

    dst = _dst_compute(tidx3.reshape(16, 128), rank3.reshape(16, 128),
                       padoff.reshape(16)).reshape(B)
    xs = _dispatch(x, dst)
    ys = _ffn(eid2.reshape(NB), valid2.reshape(NB), xs, W1, b1, W2, b2)
    y = _combine(ys, dst)

    logits3, aux = _tail(
        x, y, gate3, ln_g.reshape(1, D), ln_b.reshape(1, D), Wh1,
        bh1.reshape(1, D // 2), Wh2, bh2.reshape(1, 6), cnt, psum)

    return logits3.reshape(B, 6), tidx, aux[0, 0]


# speedup vs baseline: 2.7793x; 1.0441x over previous
import functools

import numpy as np

import jax
import jax.numpy as jnp
from jax import lax
from jax.experimental import pallas as pl
from jax.experimental.pallas import tpu as pltpu
from jax.experimental.pallas import tpu_sc as plsc

B = 2048
D = 1024
T = 8
DFF = 1024
NF = 6
VR = 16
NOPS = 2

TB = 256
NTB = B // TB
BLK = 256
NPAD = B + T * BLK
NB = NPAD // BLK
NW = 32
CHUNK = B // NW
LANES = 16


def _gelu(x):
    return 0.5 * x * (1.0 + lax.erf(x * np.float32(1.0 / np.sqrt(2.0))))


def _layernorm(x, g, b):
    m = jnp.mean(x, axis=-1, keepdims=True)
    d = x - m
    v = jnp.mean(d * d, axis=-1, keepdims=True)
    return d / jnp.sqrt(v + 1e-5) * g + b


def _frontend_body(op_ref, a_ref, b_ref, emb_ref, wop_ref, wa_ref, wb_ref,
                   bin_ref, lng_ref, lnb_ref, rw_ref, rb_ref,
                   x_ref, tidx_ref, gate_ref, dst_ref, eid_ref, valid_ref,
                   aux_ref, run_cnt, acc_psum, tidx_s, rank_s):
    i = pl.program_id(0)

    @pl.when(i == 0)
    def _():
        run_cnt[...] = jnp.zeros((1, 16), jnp.float32)
        acc_psum[...] = jnp.zeros((1, 16), jnp.float32)

    opv = op_ref[0, 0, :]
    av = a_ref[0, 0, :]
    bv = b_ref[0, 0, :]

    bf16 = jnp.bfloat16
    f32 = jnp.float32

    vi = lax.broadcasted_iota(jnp.int32, (16, 16), 0).astype(f32)
    ci = lax.broadcasted_iota(jnp.int32, (16, 16), 1)
    kk = jnp.where(ci < 6, ci, ci - 6)
    freq = jnp.left_shift(1, kk).astype(f32)
    ang = vi * (2.0 * np.pi / VR) * freq
    ftab = jnp.where(ci < 6, jnp.sin(ang),
                     jnp.where(ci < 12, jnp.cos(ang), 0.0)).astype(bf16)

    lane = lax.broadcasted_iota(jnp.int32, (TB, 16), 1)
    oh_op = (lane == opv[:, None]).astype(bf16)
    oh_a = (lane == av[:, None]).astype(bf16)
    oh_b = (lane == bv[:, None]).astype(bf16)
    feat_op = jnp.dot(oh_op, emb_ref[...].astype(bf16),
                      preferred_element_type=f32)
    fa = jnp.dot(oh_a, ftab, preferred_element_type=f32)
    fb = jnp.dot(oh_b, ftab, preferred_element_type=f32)

    x0 = (jnp.dot(feat_op.astype(bf16), wop_ref[...].astype(bf16),
                  preferred_element_type=f32)
          + jnp.dot(fa.astype(bf16), wa_ref[...].astype(bf16),
                    preferred_element_type=f32)
          + jnp.dot(fb.astype(bf16), wb_ref[...].astype(bf16),
                    preferred_element_type=f32)
          + bin_ref[...])
    x = _layernorm(x0, lng_ref[...], lnb_ref[...])
    x_ref[...] = x

    rl = jnp.dot(x.astype(bf16), rw_ref[...].astype(bf16),
                 preferred_element_type=f32) + rb_ref[...]
    rl = jnp.where(lane < T, rl, -1e30)
    mx = jnp.max(rl, axis=1, keepdims=True)
    ex = jnp.exp(rl - mx)
    probs = ex / jnp.sum(ex, axis=1, keepdims=True)
    gate = jnp.max(probs, axis=1)
    ismax = probs == gate[:, None]
    tidx = jnp.min(jnp.where(ismax, lane, T), axis=1)
    gate_ref[0, 0, :] = gate
    tidx_ref[0, 0, :] = tidx
    tidx_s[pl.ds(i, 1), :] = tidx[None, :]

    oh_e = (lane == tidx[:, None]).astype(jnp.float32)
    tri = (lax.broadcasted_iota(jnp.int32, (TB, TB), 0)
           > lax.broadcasted_iota(jnp.int32, (TB, TB), 1)).astype(jnp.float32)
    cum_excl = jnp.dot(tri, oh_e, preferred_element_type=jnp.float32)
    rank = jnp.sum((cum_excl + run_cnt[...]) * oh_e, axis=1)
    rank_s[pl.ds(i, 1), :] = rank[None, :]

    run_cnt[...] = run_cnt[...] + jnp.sum(oh_e, axis=0, keepdims=True)
    acc_psum[...] = acc_psum[...] + jnp.sum(probs, axis=0, keepdims=True)

    @pl.when(i == NTB - 1)
    def _():
        cntf = run_cnt[...]
        nblk = jnp.floor((cntf + (BLK - 1)) * (1.0 / BLK))
        ltri = (lax.broadcasted_iota(jnp.int32, (16, 16), 0)
                <= lax.broadcasted_iota(jnp.int32, (16, 16), 1)
                ).astype(jnp.float32)
        incl = jnp.dot(nblk, ltri, preferred_element_type=jnp.float32)
        padf = (incl - nblk) * BLK
        jf = lax.broadcasted_iota(jnp.int32, (16, 16), 0).astype(jnp.float32)
        cmp = (jf >= incl).astype(jnp.float32)
        eid_ref[...] = jnp.minimum(
            jnp.sum(cmp, axis=1, keepdims=True), T - 1).astype(jnp.int32)
        valid_ref[...] = (1.0 - cmp[:, T - 1:T]).astype(jnp.int32)

        acc = rank_s[...]
        tmat = tidx_s[...]
        col16 = lax.broadcasted_iota(jnp.int32, (16, 1), 0)
        for e in range(T):
            sel = (col16 == e).astype(f32)
            pe = jnp.dot(padf, sel, preferred_element_type=f32)
            acc = acc + jnp.where(tmat == e, pe, 0.0)
        dst_ref[...] = acc.astype(jnp.int32)

        aux_ref[...] = (T / (B * B)) * jnp.sum(
            run_cnt[...] * acc_psum[...], axis=1, keepdims=True)


def _frontend(op3, a3, b3, emb16, wop, wa, wb, bin2, lnig2, lnib2, rw16, rb16):
    f32 = jnp.float32
    return pl.pallas_call(
        _frontend_body,
        grid=(NTB,),
        in_specs=[
            pl.BlockSpec((1, 1, TB), lambda i: (i, 0, 0)),
            pl.BlockSpec((1, 1, TB), lambda i: (i, 0, 0)),
            pl.BlockSpec((1, 1, TB), lambda i: (i, 0, 0)),
            pl.BlockSpec((16, D // 4), lambda i: (0, 0)),
            pl.BlockSpec((D // 4, D), lambda i: (0, 0)),
            pl.BlockSpec((16, D), lambda i: (0, 0)),
            pl.BlockSpec((16, D), lambda i: (0, 0)),
            pl.BlockSpec((1, D), lambda i: (0, 0)),
            pl.BlockSpec((1, D), lambda i: (0, 0)),
            pl.BlockSpec((1, D), lambda i: (0, 0)),
            pl.BlockSpec((D, 16), lambda i: (0, 0)),
            pl.BlockSpec((1, 16), lambda i: (0, 0)),
        ],
        out_specs=[
            pl.BlockSpec((TB, D), lambda i: (i, 0)),
            pl.BlockSpec((1, 1, TB), lambda i: (i, 0, 0)),
            pl.BlockSpec((1, 1, TB), lambda i: (i, 0, 0)),
            pl.BlockSpec((NTB, TB), lambda i: (0, 0)),
            pl.BlockSpec((16, 1), lambda i: (0, 0)),
            pl.BlockSpec((16, 1), lambda i: (0, 0)),
            pl.BlockSpec((1, 1), lambda i: (0, 0)),
        ],
        out_shape=[
            jax.ShapeDtypeStruct((B, D), f32),
            jax.ShapeDtypeStruct((NTB, 1, TB), jnp.int32),
            jax.ShapeDtypeStruct((NTB, 1, TB), f32),
            jax.ShapeDtypeStruct((NTB, TB), jnp.int32),
            jax.ShapeDtypeStruct((16, 1), jnp.int32),
            jax.ShapeDtypeStruct((16, 1), jnp.int32),
            jax.ShapeDtypeStruct((1, 1), f32),
        ],
        scratch_shapes=[
            pltpu.VMEM((1, 16), f32),
            pltpu.VMEM((1, 16), f32),
            pltpu.VMEM((NTB, TB), jnp.int32),
            pltpu.VMEM((NTB, TB), f32),
        ],
    )(op3, a3, b3, emb16, wop, wa, wb, bin2, lnig2, lnib2, rw16, rb16)


def _dispatch_body(x_hbm, dst_hbm, xs_hbm, dst_v, xv, sem):
    wid = lax.axis_index("s") * 2 + lax.axis_index("c")
    base = wid * CHUNK
    pltpu.sync_copy(dst_hbm.at[pl.ds(base, CHUNK)], dst_v)
    pltpu.sync_copy(x_hbm.at[pl.ds(base, CHUNK), :], xv)
    pltpu.async_copy(xv, xs_hbm.at[dst_v], sem).wait()


def _dispatch(x, dst):
    return pl.kernel(
        _dispatch_body,
        out_type=jax.ShapeDtypeStruct((NPAD, D), jnp.float32),
        mesh=plsc.VectorSubcoreMesh(core_axis_name="c", subcore_axis_name="s", num_cores=2, num_subcores=16),
        scratch_types=[
            pltpu.VMEM((CHUNK,), jnp.int32),
            pltpu.VMEM((CHUNK, D), jnp.float32),
            pltpu.SemaphoreType.DMA,
        ],
    )(x, dst)


def _ffn_body(eid_ref, valid_ref, xs_ref, w1_ref, b1_ref, w2_ref, b2_ref,
              ys_ref):
    j = pl.program_id(0)

    @pl.when(valid_ref[j] == 1)
    def _():
        x = xs_ref[...]
        h = jnp.dot(x, w1_ref[0], preferred_element_type=jnp.float32) + b1_ref[0]
        h = _gelu(h)
        y = jnp.dot(h, w2_ref[0], preferred_element_type=jnp.float32) + b2_ref[0]
        ys_ref[...] = y


def _ffn(eid, valid, xs, W1, b1, W2, b2):
    grid_spec = pltpu.PrefetchScalarGridSpec(
        num_scalar_prefetch=2,
        grid=(NB,),
        in_specs=[
            pl.BlockSpec((BLK, D), lambda j, eid_ref, valid_ref: (j, 0)),
            pl.BlockSpec((1, D, DFF),
                         lambda j, eid_ref, valid_ref: (eid_ref[j], 0, 0)),
            pl.BlockSpec((1, 1, DFF),
                         lambda j, eid_ref, valid_ref: (eid_ref[j], 0, 0)),
            pl.BlockSpec((1, DFF, D),
                         lambda j, eid_ref, valid_ref: (eid_ref[j], 0, 0)),
            pl.BlockSpec((1, 1, D),
                         lambda j, eid_ref, valid_ref: (eid_ref[j], 0, 0)),
        ],
        out_specs=pl.BlockSpec((BLK, D), lambda j, eid_ref, valid_ref: (j, 0)),
    )
    return pl.pallas_call(
        _ffn_body,
        grid_spec=grid_spec,
        out_shape=jax.ShapeDtypeStruct((NPAD, D), jnp.float32),
    )(eid, valid, xs, W1, b1.reshape(T, 1, DFF), W2, b2.reshape(T, 1, D))


def _combine_body(ys_hbm, dst_hbm, y_hbm, dst_v, yv, sem):
    wid = lax.axis_index("s") * 2 + lax.axis_index("c")
    base = wid * CHUNK
    pltpu.sync_copy(dst_hbm.at[pl.ds(base, CHUNK)], dst_v)
    pltpu.async_copy(ys_hbm.at[dst_v], yv, sem).wait()
    pltpu.sync_copy(yv, y_hbm.at[pl.ds(base, CHUNK), :])


def _combine(ys, dst):
    return pl.kernel(
        _combine_body,
        out_type=jax.ShapeDtypeStruct((B, D), jnp.float32),
        mesh=plsc.VectorSubcoreMesh(core_axis_name="c", subcore_axis_name="s", num_cores=2, num_subcores=16),
        scratch_types=[
            pltpu.VMEM((CHUNK,), jnp.int32),
            pltpu.VMEM((CHUNK, D), jnp.float32),
            pltpu.SemaphoreType.DMA,
        ],
    )(ys, dst)


def _tail_body(x_ref, y_ref, gate_ref, lng_ref, lnb_ref, wh1_ref, bh1_ref,
               wh2_ref, bh2_ref, logits_ref):
    g = gate_ref[0, 0, :][:, None]
    z = _layernorm(x_ref[...] + y_ref[...] * g, lng_ref[...], lnb_ref[...])
    h = _gelu(jnp.dot(z, wh1_ref[...], preferred_element_type=jnp.float32)
              + bh1_ref[...])
    logits_ref[0] = jnp.dot(h, wh2_ref[...],
                            preferred_element_type=jnp.float32) + bh2_ref[...]


def _tail(x, y, gate3, lng2, lnb2, Wh1, bh1_2, Wh2, bh2_2):
    f32 = jnp.float32
    return pl.pallas_call(
        _tail_body,
        grid=(NTB,),
        in_specs=[
            pl.BlockSpec((TB, D), lambda i: (i, 0)),
            pl.BlockSpec((TB, D), lambda i: (i, 0)),
            pl.BlockSpec((1, 1, TB), lambda i: (i, 0, 0)),
            pl.BlockSpec((1, D), lambda i: (0, 0)),
            pl.BlockSpec((1, D), lambda i: (0, 0)),
            pl.BlockSpec((D, D // 2), lambda i: (0, 0)),
            pl.BlockSpec((1, D // 2), lambda i: (0, 0)),
            pl.BlockSpec((D // 2, 6), lambda i: (0, 0)),
            pl.BlockSpec((1, 6), lambda i: (0, 0)),
        ],
        out_specs=pl.BlockSpec((1, TB, 6), lambda i: (i, 0, 0)),
        out_shape=jax.ShapeDtypeStruct((NTB, TB, 6), f32),
    )(x, y, gate3, lng2, lnb2, Wh1, bh1_2, Wh2, bh2_2)


def kernel(op, a, b, op_embed, W_in, b_in, ln_in_g, ln_in_b, router_W,
           router_b, W1, b1, W2, b2, ln_g, ln_b, Wh1, bh1, Wh2, bh2):
    i32 = jnp.int32
    op3 = op.astype(i32).reshape(NTB, 1, TB)
    a3 = a.astype(i32).reshape(NTB, 1, TB)
    b3 = b.astype(i32).reshape(NTB, 1, TB)
    emb16 = jnp.pad(op_embed, ((0, 16 - NOPS), (0, 0)))
    wop = W_in[:D // 4]
    wa = jnp.pad(W_in[D // 4:D // 4 + 2 * NF], ((0, 16 - 2 * NF), (0, 0)))
    wb = jnp.pad(W_in[D // 4 + 2 * NF:], ((0, 16 - 2 * NF), (0, 0)))
    rw16 = jnp.pad(router_W, ((0, 0), (0, 16 - T)))
    rb16 = jnp.pad(router_b, (0, 16 - T)).reshape(1, 16)

    (x, tidx3, gate3, dst2, eid2, valid2, aux) = _frontend(
        op3, a3, b3, emb16, wop, wa, wb, b_in.reshape(1, D),
        ln_in_g.reshape(1, D), ln_in_b.reshape(1, D), rw16, rb16)

    tidx = tidx3.reshape(B)
    dst = dst2.reshape(B)

    xs = _dispatch(x, dst)
    ys = _ffn(eid2.reshape(NB), valid2.reshape(NB), xs, W1, b1, W2, b2)
    y = _combine(ys, dst)

    logits3 = _tail(
        x, y, gate3, ln_g.reshape(1, D), ln_b.reshape(1, D), Wh1,
        bh1.reshape(1, D // 2), Wh2, bh2.reshape(1, 6))

    return logits3.reshape(B, 6), tidx, aux[0, 0]
